# baseline (device time: 777234 ns/iter reference)
import jax
import jax.numpy as jnp
from jax import lax
from jax.experimental import pallas as pl
from jax.experimental.pallas import tpu as pltpu

N_DEV = 32
GELU_C = 0.7978845608028654


def _gelu(y):
    return 0.5 * y * (1.0 + jnp.tanh(GELU_C * (y + 0.044715 * y * y * y)))


def kernel(x, w_mat):
    m_per, k = x.shape
    _, n_per = w_mat.shape

    def body(x_ref, w_ref, out_ref, comm_ref, send_sems, recv_sems):
        my_pos = lax.axis_index("i")
        left = (my_pos - 1) % N_DEV
        right = (my_pos + 1) % N_DEV

        barrier_sem = pltpu.get_barrier_semaphore()
        for nbr in [left, right]:
            pl.semaphore_signal(
                barrier_sem, inc=1,
                device_id=(nbr,), device_id_type=pl.DeviceIdType.MESH,
            )
        pl.semaphore_wait(barrier_sem, 2)

        comm_ref[0, :, :] = x_ref[:, :]
        own = jnp.dot(x_ref[:, :], w_ref[:, :],
                      preferred_element_type=jnp.float32)
        out_ref[pl.ds(my_pos * m_per, m_per), :] = _gelu(own)

        for h in range(N_DEV - 1):
            send_slot = h % 2
            recv_slot = (h + 1) % 2
            rdma = pltpu.make_async_remote_copy(
                src_ref=comm_ref.at[send_slot],
                dst_ref=comm_ref.at[recv_slot],
                send_sem=send_sems.at[send_slot],
                recv_sem=recv_sems.at[recv_slot],
                device_id=(right,),
                device_id_type=pl.DeviceIdType.MESH,
            )
            rdma.start()
            rdma.wait()

            origin = (my_pos - h - 1) % N_DEV
            y = jnp.dot(comm_ref[recv_slot, :, :], w_ref[:, :],
                        preferred_element_type=jnp.float32)
            out_ref[pl.ds(origin * m_per, m_per), :] = _gelu(y)

    return pl.pallas_call(
        body,
        out_shape=jax.ShapeDtypeStruct((N_DEV * m_per, n_per), jnp.float32),
        in_specs=[
            pl.BlockSpec(memory_space=pltpu.VMEM),
            pl.BlockSpec(memory_space=pltpu.VMEM),
        ],
        out_specs=pl.BlockSpec(memory_space=pltpu.VMEM),
        scratch_shapes=[
            pltpu.VMEM((2, m_per, k), x.dtype),
            pltpu.SemaphoreType.DMA((2,)),
            pltpu.SemaphoreType.DMA((2,)),
        ],
        compiler_params=pltpu.CompilerParams(collective_id=0),
    )(x, w_mat)


# device time: 439383 ns/iter; 1.7689x vs baseline; 1.7689x over previous
import jax
import jax.numpy as jnp
from jax import lax
from jax.experimental import pallas as pl
from jax.experimental.pallas import tpu as pltpu

N_DEV = 32
GELU_C = 0.7978845608028654

_PLANE = [(0, 0), (1, 0), (1, 1), (0, 1), (0, 2), (1, 2), (1, 3), (0, 3)]
_MESH_IDX = {}
for _z in range(4):
    for _k, (_x, _y) in enumerate(_PLANE):
        _MESH_IDX[(_x, _y, _z)] = _z * 8 + _k

_C = [(0, 0), (0, 1), (0, 2), (0, 3), (1, 3), (1, 2), (1, 1), (2, 1),
      (2, 2), (2, 3), (3, 3), (3, 2), (3, 1), (3, 0), (2, 0), (1, 0)]
_RING_COORDS = [(0, y, z) for (y, z) in _C] + [(1, y, z) for (y, z) in reversed(_C)]
RING = [_MESH_IDX[c] for c in _RING_COORDS]
POS = [0] * N_DEV
for _p, _m in enumerate(RING):
    POS[_m] = _p

N_CW = 16
N_CCW = 15


def _gelu(y):
    return 0.5 * y * (1.0 + jnp.tanh(GELU_C * (y + 0.044715 * y * y * y)))


def kernel(x, w_mat):
    m_per, k = x.shape
    _, n_per = w_mat.shape

    ring = jnp.asarray(RING, dtype=jnp.int32)
    pos = jnp.asarray(POS, dtype=jnp.int32)
    my = lax.axis_index("i").astype(jnp.int32)
    my_ring = pos[my]
    right = ring[(my_ring + 1) % N_DEV]
    left = ring[(my_ring - 1) % N_DEV]
    origins_cw = jnp.stack(
        [ring[(my_ring - h - 1) % N_DEV] for h in range(N_CW)])
    origins_ccw = jnp.stack(
        [ring[(my_ring + h + 1) % N_DEV] for h in range(N_CCW)])
    meta = jnp.concatenate(
        [jnp.stack([right, left]), origins_cw, origins_ccw]).astype(jnp.int32)

    def body(meta_ref, x_ref, w_ref, out_ref,
             cw_comm, ccw_comm, cw_send, cw_recv, ccw_send, ccw_recv):
        rgt = meta_ref[0]
        lft = meta_ref[1]
        my_pos = lax.axis_index("i")

        barrier_sem = pltpu.get_barrier_semaphore()
        for nbr in [lft, rgt]:
            pl.semaphore_signal(
                barrier_sem, inc=1,
                device_id=(nbr,), device_id_type=pl.DeviceIdType.MESH,
            )
        pl.semaphore_wait(barrier_sem, 2)

        cw_comm[0, :, :] = x_ref[:, :]
        ccw_comm[0, :, :] = x_ref[:, :]
        own = jnp.dot(x_ref[:, :], w_ref[:, :],
                      preferred_element_type=jnp.float32)
        out_ref[pl.ds(my_pos * m_per, m_per), :] = _gelu(own)

        for h in range(N_CW):
            s, r = h % 2, (h + 1) % 2
            cw_rdma = pltpu.make_async_remote_copy(
                src_ref=cw_comm.at[s],
                dst_ref=cw_comm.at[r],
                send_sem=cw_send.at[s],
                recv_sem=cw_recv.at[r],
                device_id=(rgt,),
                device_id_type=pl.DeviceIdType.MESH,
            )
            cw_rdma.start()
            if h < N_CCW:
                ccw_rdma = pltpu.make_async_remote_copy(
                    src_ref=ccw_comm.at[s],
                    dst_ref=ccw_comm.at[r],
                    send_sem=ccw_send.at[s],
                    recv_sem=ccw_recv.at[r],
                    device_id=(lft,),
                    device_id_type=pl.DeviceIdType.MESH,
                )
                ccw_rdma.start()

            cw_rdma.wait()
            y = jnp.dot(cw_comm[r, :, :], w_ref[:, :],
                        preferred_element_type=jnp.float32)
            out_ref[pl.ds(meta_ref[2 + h] * m_per, m_per), :] = _gelu(y)

            if h < N_CCW:
                ccw_rdma.wait()
                y = jnp.dot(ccw_comm[r, :, :], w_ref[:, :],
                            preferred_element_type=jnp.float32)
                out_ref[pl.ds(meta_ref[2 + N_CW + h] * m_per, m_per), :] = \
                    _gelu(y)

    return pl.pallas_call(
        body,
        out_shape=jax.ShapeDtypeStruct((N_DEV * m_per, n_per), jnp.float32),
        in_specs=[
            pl.BlockSpec(memory_space=pltpu.SMEM),
            pl.BlockSpec(memory_space=pltpu.VMEM),
            pl.BlockSpec(memory_space=pltpu.VMEM),
        ],
        out_specs=pl.BlockSpec(memory_space=pltpu.VMEM),
        scratch_shapes=[
            pltpu.VMEM((2, m_per, k), x.dtype),
            pltpu.VMEM((2, m_per, k), x.dtype),
            pltpu.SemaphoreType.DMA((2,)),
            pltpu.SemaphoreType.DMA((2,)),
            pltpu.SemaphoreType.DMA((2,)),
            pltpu.SemaphoreType.DMA((2,)),
        ],
        compiler_params=pltpu.CompilerParams(collective_id=0),
    )(meta, x, w_mat)


# device time: 420819 ns/iter; 1.8470x vs baseline; 1.0441x over previous
import jax
import jax.numpy as jnp
from jax import lax
from jax.experimental import pallas as pl
from jax.experimental.pallas import tpu as pltpu

N_DEV = 32
N_SLOT = 4
GELU_C = 0.7978845608028654

_PLANE = [(0, 0), (1, 0), (1, 1), (0, 1), (0, 2), (1, 2), (1, 3), (0, 3)]
_MESH_IDX = {}
for _z in range(4):
    for _k, (_x, _y) in enumerate(_PLANE):
        _MESH_IDX[(_x, _y, _z)] = _z * 8 + _k

_C = [(0, 0), (0, 1), (0, 2), (0, 3), (1, 3), (1, 2), (1, 1), (2, 1),
      (2, 2), (2, 3), (3, 3), (3, 2), (3, 1), (3, 0), (2, 0), (1, 0)]
_RING_COORDS = [(0, y, z) for (y, z) in _C] + [(1, y, z) for (y, z) in reversed(_C)]
RING = [_MESH_IDX[c] for c in _RING_COORDS]
POS = [0] * N_DEV
for _p, _m in enumerate(RING):
    POS[_m] = _p

N_CW = 16
N_CCW = 15


def _gelu(y):
    return 0.5 * y * (1.0 + jnp.tanh(GELU_C * (y + 0.044715 * y * y * y)))


def kernel(x, w_mat):
    m_per, k = x.shape
    _, n_per = w_mat.shape

    ring = jnp.asarray(RING, dtype=jnp.int32)
    pos = jnp.asarray(POS, dtype=jnp.int32)
    my = lax.axis_index("i").astype(jnp.int32)
    my_ring = pos[my]
    right = ring[(my_ring + 1) % N_DEV]
    left = ring[(my_ring - 1) % N_DEV]
    origins_cw = jnp.stack(
        [ring[(my_ring - h - 1) % N_DEV] for h in range(N_CW)])
    origins_ccw = jnp.stack(
        [ring[(my_ring + h + 1) % N_DEV] for h in range(N_CCW)])
    meta = jnp.concatenate(
        [jnp.stack([right, left]), origins_cw, origins_ccw]).astype(jnp.int32)

    def body(meta_ref, x_ref, w_ref, out_ref,
             cw_comm, ccw_comm, cw_send, cw_recv, ccw_send, ccw_recv,
             cw_credit, ccw_credit):
        rgt = meta_ref[0]
        lft = meta_ref[1]
        my_pos = lax.axis_index("i")

        def make_cw(h):
            return pltpu.make_async_remote_copy(
                src_ref=cw_comm.at[h % N_SLOT],
                dst_ref=cw_comm.at[(h + 1) % N_SLOT],
                send_sem=cw_send.at[h % N_SLOT],
                recv_sem=cw_recv.at[(h + 1) % N_SLOT],
                device_id=(rgt,),
                device_id_type=pl.DeviceIdType.MESH,
            )

        def make_ccw(h):
            return pltpu.make_async_remote_copy(
                src_ref=ccw_comm.at[h % N_SLOT],
                dst_ref=ccw_comm.at[(h + 1) % N_SLOT],
                send_sem=ccw_send.at[h % N_SLOT],
                recv_sem=ccw_recv.at[(h + 1) % N_SLOT],
                device_id=(lft,),
                device_id_type=pl.DeviceIdType.MESH,
            )

        barrier_sem = pltpu.get_barrier_semaphore()
        for nbr in [lft, rgt]:
            pl.semaphore_signal(
                barrier_sem, inc=1,
                device_id=(nbr,), device_id_type=pl.DeviceIdType.MESH,
            )
        pl.semaphore_wait(barrier_sem, 2)

        cw_comm[0, :, :] = x_ref[:, :]
        ccw_comm[0, :, :] = x_ref[:, :]
        make_cw(0).start()
        make_ccw(0).start()

        own = jnp.dot(x_ref[:, :], w_ref[:, :],
                      preferred_element_type=jnp.float32)
        out_ref[pl.ds(my_pos * m_per, m_per), :] = _gelu(own)

        for h in range(N_CW):
            r = (h + 1) % N_SLOT

            cw_rdma = make_cw(h)
            cw_rdma.wait_recv()
            if h + 1 <= N_CW - 1:
                cw_rdma.wait_send()
                if h <= N_CW - 4:
                    pl.semaphore_signal(
                        cw_credit, inc=1,
                        device_id=(lft,),
                        device_id_type=pl.DeviceIdType.MESH,
                    )
                if h + 1 >= 3:
                    pl.semaphore_wait(cw_credit, 1)
                make_cw(h + 1).start()

            has_ccw = h < N_CCW
            if has_ccw:
                ccw_rdma = make_ccw(h)
                ccw_rdma.wait_recv()
                if h + 1 <= N_CCW - 1:
                    ccw_rdma.wait_send()
                    if h <= N_CCW - 4:
                        pl.semaphore_signal(
                            ccw_credit, inc=1,
                            device_id=(rgt,),
                            device_id_type=pl.DeviceIdType.MESH,
                        )
                    if h + 1 >= 3:
                        pl.semaphore_wait(ccw_credit, 1)
                    make_ccw(h + 1).start()

            y = jnp.dot(cw_comm[r, :, :], w_ref[:, :],
                        preferred_element_type=jnp.float32)
            out_ref[pl.ds(meta_ref[2 + h] * m_per, m_per), :] = _gelu(y)
            if has_ccw:
                y = jnp.dot(ccw_comm[r, :, :], w_ref[:, :],
                            preferred_element_type=jnp.float32)
                out_ref[pl.ds(meta_ref[2 + N_CW + h] * m_per, m_per), :] = \
                    _gelu(y)

        make_cw(N_CW - 1).wait_send()
        make_ccw(N_CCW - 1).wait_send()

    return pl.pallas_call(
        body,
        out_shape=jax.ShapeDtypeStruct((N_DEV * m_per, n_per), jnp.float32),
        in_specs=[
            pl.BlockSpec(memory_space=pltpu.SMEM),
            pl.BlockSpec(memory_space=pltpu.VMEM),
            pl.BlockSpec(memory_space=pltpu.VMEM),
        ],
        out_specs=pl.BlockSpec(memory_space=pltpu.VMEM),
        scratch_shapes=[
            pltpu.VMEM((N_SLOT, m_per, k), x.dtype),
            pltpu.VMEM((N_SLOT, m_per, k), x.dtype),
            pltpu.SemaphoreType.DMA((N_SLOT,)),
            pltpu.SemaphoreType.DMA((N_SLOT,)),
            pltpu.SemaphoreType.DMA((N_SLOT,)),
            pltpu.SemaphoreType.DMA((N_SLOT,)),
            pltpu.SemaphoreType.REGULAR,
            pltpu.SemaphoreType.REGULAR,
        ],
        compiler_params=pltpu.CompilerParams(collective_id=0),
    )(meta, x, w_mat)
